# baseline (speedup 1.0000x reference)
import functools

import jax
import jax.numpy as jnp
from jax import lax
from jax.experimental import pallas as pl
from jax.experimental.pallas import tpu as pltpu
from jax.experimental.pallas import tpu_sc as plsc
from jax.experimental import layout as jl

NC = 2
NS = 16
NW = NC * NS
CHUNK = 128
D = 64
L = 16


@functools.partial(jax.jit, static_argnums=(2,))
def _gather(table, idx2, H):
    mesh = plsc.VectorSubcoreMesh(
        core_axis_name="c", subcore_axis_name="s",
        num_cores=NC, num_subcores=NS)

    @functools.partial(
        pl.kernel,
        out_type=jax.ShapeDtypeStruct((H * D, NW * CHUNK), jnp.float32),
        mesh=mesh,
        scratch_types=[
            pltpu.VMEM((H, CHUNK), jnp.int32),
            pltpu.VMEM((2, CHUNK, D), jnp.float32),
            pltpu.VMEM((2, D, CHUNK), jnp.float32),
            pltpu.SemaphoreType.DMA,
            pltpu.SemaphoreType.DMA,
            pltpu.SemaphoreType.DMA,
        ],
        compiler_params=pltpu.CompilerParams(
            use_tc_tiling_on_sc=False,
            needs_layout_passes=False,
        ),
    )
    def k(table_hbm, idx_hbm, out_hbm, idx_v, rows_v, rows_t, gsem, wsem0, wsem1):
        wid = lax.axis_index("s") * NC + lax.axis_index("c")
        pltpu.sync_copy(idx_hbm.at[wid], idx_v)
        wsems = (wsem0, wsem1)
        col = wid * CHUNK

        def fire_gather(h, s):
            return pltpu.async_copy(
                table_hbm.at[idx_v.at[h]], rows_v.at[s], gsem)

        def transpose(s):
            for g in range(CHUNK // L):
                rows16 = jax.lax.iota(jnp.int32, L) + g * L
                for f in range(D):
                    cols16 = jnp.full((L,), f, jnp.int32)
                    vals = plsc.load_gather(rows_v.at[s], [rows16, cols16])
                    rows_t[s, f, pl.ds(g * L, L)] = vals

        def fire_write(h, s):
            return pltpu.async_copy(
                rows_t.at[s], out_hbm.at[pl.ds(h * D, D), pl.ds(col, CHUNK)], wsems[s])

        def drain_write(h, s):
            pltpu.make_async_copy(
                rows_t.at[s], out_hbm.at[pl.ds(h * D, D), pl.ds(col, CHUNK)], wsems[s]
            ).wait()

        @pl.loop(0, H // 2)
        def _(t):
            h0 = t * 2
            for s in range(2):
                h = h0 + s
                g = fire_gather(h, s)

                @pl.when(t > 0)
                def _():
                    drain_write(h, s)

                g.wait()
                transpose(s)
                fire_write(h, s)

        for s in range(2):
            drain_write(H - 2 + s, s)

    return k(table, idx2)


def kernel(q, table):
    Bq, H = q.shape
    idx = q.astype(jnp.int32).reshape(NW, CHUNK, H).transpose(0, 2, 1)
    t_lin = jl.with_layout_constraint(
        table, jl.Layout(major_to_minor=(1, 0), tiling=((8,),)))
    o = _gather(t_lin, idx, H)
    return o.reshape(H, D, NW * CHUNK).transpose(2, 0, 1)


# speedup vs baseline: 1.3931x; 1.3931x over previous
import functools

import jax
import jax.numpy as jnp
from jax import lax
from jax.experimental import pallas as pl
from jax.experimental.pallas import tpu as pltpu
from jax.experimental.pallas import tpu_sc as plsc
from jax.experimental import layout as jl

NC = 2
NS = 16
NW = NC * NS
CHUNK = 128
D = 64
L = 16


@functools.partial(jax.jit, static_argnums=(2,))
def _gather(table, idx2, H):
    mesh = plsc.VectorSubcoreMesh(
        core_axis_name="c", subcore_axis_name="s",
        num_cores=NC, num_subcores=NS)

    @functools.partial(
        pl.kernel,
        out_type=jax.ShapeDtypeStruct((H * D, NW * CHUNK), jnp.float32),
        mesh=mesh,
        scratch_types=[
            pltpu.VMEM((H, CHUNK), jnp.int32),
            pltpu.VMEM((2, CHUNK, D), jnp.float32),
            pltpu.VMEM((2, D, CHUNK), jnp.float32),
            pltpu.SemaphoreType.DMA,
            pltpu.SemaphoreType.DMA,
            pltpu.SemaphoreType.DMA,
        ],
        compiler_params=pltpu.CompilerParams(
            use_tc_tiling_on_sc=False,
            needs_layout_passes=False,
        ),
    )
    def k(table_hbm, idx_hbm, out_hbm, idx_v, rows_v, rows_t, gsem, wsem0, wsem1):
        wid = lax.axis_index("s") * NC + lax.axis_index("c")
        pltpu.sync_copy(idx_hbm.at[wid], idx_v)
        wsems = (wsem0, wsem1)
        col = wid * CHUNK

        def fire_gather(h, s):
            return pltpu.async_copy(
                table_hbm.at[idx_v.at[h]], rows_v.at[s], gsem)

        def transpose(s):
            W = 8
            for g in range(CHUNK // L):
                rows16 = jax.lax.iota(jnp.int32, L) + g * L
                for f0 in range(0, D, W):
                    vals = [
                        plsc.load_gather(
                            rows_v.at[s],
                            [rows16, jnp.full((L,), f0 + i, jnp.int32)])
                        for i in range(W)
                    ]
                    for i in range(W):
                        rows_t[s, f0 + i, pl.ds(g * L, L)] = vals[i]

        def fire_write(h, s):
            return pltpu.async_copy(
                rows_t.at[s], out_hbm.at[pl.ds(h * D, D), pl.ds(col, CHUNK)], wsems[s])

        def drain_write(h, s):
            pltpu.make_async_copy(
                rows_t.at[s], out_hbm.at[pl.ds(h * D, D), pl.ds(col, CHUNK)], wsems[s]
            ).wait()

        @pl.loop(0, H // 2)
        def _(t):
            h0 = t * 2
            for s in range(2):
                h = h0 + s
                g = fire_gather(h, s)

                @pl.when(t > 0)
                def _():
                    drain_write(h, s)

                g.wait()
                transpose(s)
                fire_write(h, s)

        for s in range(2):
            drain_write(H - 2 + s, s)

    return k(table, idx2)


def kernel(q, table):
    Bq, H = q.shape
    idx = q.astype(jnp.int32).reshape(NW, CHUNK, H).transpose(0, 2, 1)
    t_lin = jl.with_layout_constraint(
        table, jl.Layout(major_to_minor=(1, 0), tiling=((8,),)))
    o = _gather(t_lin, idx, H)
    return o.reshape(H, D, NW * CHUNK).transpose(2, 0, 1)


# speedup vs baseline: 1.5021x; 1.0783x over previous
import functools

import jax
import jax.numpy as jnp
from jax import lax
from jax.experimental import pallas as pl
from jax.experimental.pallas import tpu as pltpu
from jax.experimental.pallas import tpu_sc as plsc
from jax.experimental import layout as jl

NC = 2
NS = 16
NW = NC * NS
CHUNK = 128
D = 64
L = 16


@functools.partial(jax.jit, static_argnums=(2,))
def _gather(table, idx2, H):
    mesh = plsc.VectorSubcoreMesh(
        core_axis_name="c", subcore_axis_name="s",
        num_cores=NC, num_subcores=NS)

    @functools.partial(
        pl.kernel,
        out_type=jax.ShapeDtypeStruct((H * D, NW * CHUNK), jnp.float32),
        mesh=mesh,
        scratch_types=[
            pltpu.VMEM((H, CHUNK), jnp.int32),
            pltpu.VMEM((4, CHUNK, D), jnp.float32),
            pltpu.VMEM((4, D, CHUNK), jnp.float32),
            pltpu.SemaphoreType.DMA,
            pltpu.SemaphoreType.DMA,
            pltpu.SemaphoreType.DMA,
            pltpu.SemaphoreType.DMA,
            pltpu.SemaphoreType.DMA,
        ],
        compiler_params=pltpu.CompilerParams(
            use_tc_tiling_on_sc=False,
            needs_layout_passes=False,
        ),
    )
    def k(table_hbm, idx_hbm, out_hbm, idx_v, rows_v, rows_t, gsem,
          wsem0, wsem1, wsem2, wsem3):
        wid = lax.axis_index("s") * NC + lax.axis_index("c")
        pltpu.sync_copy(idx_hbm.at[wid], idx_v)
        wsems = (wsem0, wsem1, wsem2, wsem3)
        col = wid * CHUNK

        def fire_gather(h, s):
            return pltpu.async_copy(
                table_hbm.at[idx_v.at[h]], rows_v.at[s], gsem)

        def transpose(s):
            W = 8
            for g in range(CHUNK // L):
                rows16 = jax.lax.iota(jnp.int32, L) + g * L
                for f0 in range(0, D, W):
                    vals = [
                        plsc.load_gather(
                            rows_v.at[s],
                            [rows16, jnp.full((L,), f0 + i, jnp.int32)])
                        for i in range(W)
                    ]
                    for i in range(W):
                        rows_t[s, f0 + i, pl.ds(g * L, L)] = vals[i]

        def fire_write(h, s):
            return pltpu.async_copy(
                rows_t.at[s], out_hbm.at[pl.ds(h * D, D), pl.ds(col, CHUNK)], wsems[s])

        def drain_write(h, s):
            pltpu.make_async_copy(
                rows_t.at[s], out_hbm.at[pl.ds(h * D, D), pl.ds(col, CHUNK)], wsems[s]
            ).wait()

        NB = 4

        @pl.loop(0, H // NB)
        def _(t):
            h0 = t * NB
            descs = [fire_gather(h0 + s, s) for s in range(NB)]
            for s in range(NB):
                h = h0 + s

                @pl.when(t > 0)
                def _():
                    drain_write(h, s)

                descs[s].wait()
                transpose(s)
                fire_write(h, s)

        for s in range(NB):
            drain_write(H - NB + s, s)

    return k(table, idx2)


def kernel(q, table):
    Bq, H = q.shape
    idx = q.astype(jnp.int32).reshape(NW, CHUNK, H).transpose(0, 2, 1)
    t_lin = jl.with_layout_constraint(
        table, jl.Layout(major_to_minor=(1, 0), tiling=((8,),)))
    o = _gather(t_lin, idx, H)
    return o.reshape(H, D, NW * CHUNK).transpose(2, 0, 1)
